# baseline (speedup 1.0000x reference)
import jax
import jax.numpy as jnp
from jax import lax
from jax.experimental import pallas as pl
from jax.experimental.pallas import tpu as pltpu

_B, _C, _H, _W = 4, 384, 224, 224
_HW = _H * _W
_K = int(0.7 * _C)

_POOL_CC = 32
_APPLY_CC = 32


def _pool_body(feat_ref, out_ref):
    out_ref[0, 0, 0, :] = jnp.sum(feat_ref[0], axis=(1, 2))


def _mlp_mask_body(pooled_ref, w1_ref, b1_ref, w2_ref, b2_ref,
                   soft_ref, mask_ref):
    pooled = pooled_ref[...] * (1.0 / _HW)
    h = lax.dot_general(pooled, w1_ref[...],
                        (((1,), (1,)), ((), ())),
                        preferred_element_type=jnp.float32)
    h = jnp.maximum(h + b1_ref[...][None, :], 0.0)
    z = lax.dot_general(h, w2_ref[...],
                        (((1,), (1,)), ((), ())),
                        preferred_element_type=jnp.float32)
    soft = jax.nn.sigmoid(z + b2_ref[...][None, :])
    soft_ref[...] = soft
    vi = soft[:, :, None]
    vj = soft[:, None, :]
    ii = lax.broadcasted_iota(jnp.int32, (_B, _C, _C), 1)
    jj = lax.broadcasted_iota(jnp.int32, (_B, _C, _C), 2)
    beats = (vj > vi) | ((vj == vi) & (jj < ii))
    rank = jnp.sum(beats.astype(jnp.int32), axis=2)
    mask_ref[...] = (rank < _K).astype(jnp.float32)


def _apply_body(mask_ref, feat_ref, causal_ref, noncausal_ref):
    jb = pl.program_id(0)
    jc = pl.program_id(1)
    for i in range(_APPLY_CC):
        m = mask_ref[jb, jc * _APPLY_CC + i]
        f = feat_ref[0, i]
        c = f * m
        causal_ref[0, i] = c
        noncausal_ref[0, i] = f - c


@jax.jit
def kernel(feat, w1, b1, w2, b2):
    npool = _C // _POOL_CC
    psums = pl.pallas_call(
        _pool_body,
        grid=(_B, npool),
        in_specs=[
            pl.BlockSpec((1, _POOL_CC, _H, _W), lambda b, j: (b, j, 0, 0)),
        ],
        out_specs=pl.BlockSpec((1, 1, 1, _POOL_CC), lambda b, j: (b, j, 0, 0)),
        out_shape=jax.ShapeDtypeStruct((_B, npool, 1, _POOL_CC), jnp.float32),
    )(feat)

    pooled = psums.reshape(_B, _C)

    soft_mask, mask = pl.pallas_call(
        _mlp_mask_body,
        out_shape=[
            jax.ShapeDtypeStruct((_B, _C), jnp.float32),
            jax.ShapeDtypeStruct((_B, _C), jnp.float32),
        ],
    )(pooled, w1, b1, w2, b2)

    napply = _C // _APPLY_CC
    causal, noncausal = pl.pallas_call(
        _apply_body,
        grid=(_B, napply),
        in_specs=[
            pl.BlockSpec(memory_space=pltpu.SMEM),
            pl.BlockSpec((1, _APPLY_CC, _H, _W), lambda b, j: (b, j, 0, 0)),
        ],
        out_specs=[
            pl.BlockSpec((1, _APPLY_CC, _H, _W), lambda b, j: (b, j, 0, 0)),
            pl.BlockSpec((1, _APPLY_CC, _H, _W), lambda b, j: (b, j, 0, 0)),
        ],
        out_shape=[
            jax.ShapeDtypeStruct((_B, _C, _H, _W), jnp.float32),
            jax.ShapeDtypeStruct((_B, _C, _H, _W), jnp.float32),
        ],
    )(mask, feat)

    mask4 = mask.reshape(_B, _C, 1, 1)
    return (causal, noncausal, mask4, soft_mask)


# speedup vs baseline: 3.5387x; 3.5387x over previous
import jax
import jax.numpy as jnp
from jax import lax
from jax.experimental import pallas as pl

_B, _C, _H, _W = 4, 384, 224, 224
_HW = _H * _W
_K = int(0.7 * _C)

_HH = 16
_NH = _H // _HH


def _pool_body(x_ref, out_ref):
    out_ref[0, 0, 0, :] = jnp.sum(x_ref[0], axis=(0, 1))


def _mlp_mask_body(psums_ref, w1_ref, b1_ref, w2_ref, b2_ref,
                   soft_ref, mask_ref):
    pooled = jnp.sum(psums_ref[...], axis=(1, 2)) * (1.0 / _HW)
    h = lax.dot_general(pooled, w1_ref[...],
                        (((1,), (1,)), ((), ())),
                        preferred_element_type=jnp.float32)
    h = jnp.maximum(h + b1_ref[...][None, :], 0.0)
    z = lax.dot_general(h, w2_ref[...],
                        (((1,), (1,)), ((), ())),
                        preferred_element_type=jnp.float32)
    soft = jax.nn.sigmoid(z + b2_ref[...][None, :])
    soft_ref[...] = soft
    vi = soft[:, :, None]
    vj = soft[:, None, :]
    ii = lax.broadcasted_iota(jnp.int32, (_B, _C, _C), 1)
    jj = lax.broadcasted_iota(jnp.int32, (_B, _C, _C), 2)
    beats = (vj > vi) | ((vj == vi) & (jj < ii))
    rank = jnp.sum(beats.astype(jnp.int32), axis=2)
    mask_ref[...] = (rank < _K).astype(jnp.float32)


def _apply_body(mask_ref, x_ref, causal_ref, noncausal_ref):
    m = mask_ref[0, 0, :][None, None, :]
    x = x_ref[0]
    c = x * m
    causal_ref[0] = c
    noncausal_ref[0] = x - c


@jax.jit
def kernel(feat, w1, b1, w2, b2):
    x = jnp.transpose(feat, (0, 2, 3, 1))

    psums = pl.pallas_call(
        _pool_body,
        grid=(_B, _NH),
        in_specs=[
            pl.BlockSpec((1, _HH, _W, _C), lambda b, h: (b, h, 0, 0)),
        ],
        out_specs=pl.BlockSpec((1, 1, 1, _C), lambda b, h: (b, h, 0, 0)),
        out_shape=jax.ShapeDtypeStruct((_B, _NH, 1, _C), jnp.float32),
    )(x)

    soft_mask, mask = pl.pallas_call(
        _mlp_mask_body,
        out_shape=[
            jax.ShapeDtypeStruct((_B, _C), jnp.float32),
            jax.ShapeDtypeStruct((_B, _C), jnp.float32),
        ],
    )(psums, w1, b1, w2, b2)

    causal, noncausal = pl.pallas_call(
        _apply_body,
        grid=(_B, _NH),
        in_specs=[
            pl.BlockSpec((1, 1, _C), lambda b, h: (b, 0, 0)),
            pl.BlockSpec((1, _HH, _W, _C), lambda b, h: (b, h, 0, 0)),
        ],
        out_specs=[
            pl.BlockSpec((1, _HH, _W, _C), lambda b, h: (b, h, 0, 0)),
            pl.BlockSpec((1, _HH, _W, _C), lambda b, h: (b, h, 0, 0)),
        ],
        out_shape=[
            jax.ShapeDtypeStruct((_B, _H, _W, _C), jnp.float32),
            jax.ShapeDtypeStruct((_B, _H, _W, _C), jnp.float32),
        ],
    )(mask.reshape(_B, 1, _C), x)

    causal = jnp.transpose(causal, (0, 3, 1, 2))
    noncausal = jnp.transpose(noncausal, (0, 3, 1, 2))
    mask4 = mask.reshape(_B, _C, 1, 1)
    return (causal, noncausal, mask4, soft_mask)


# speedup vs baseline: 3.5948x; 1.0159x over previous
import jax
import jax.numpy as jnp
from jax import lax
from jax.experimental import pallas as pl

_B, _C, _H, _W = 4, 384, 224, 224
_HW = _H * _W
_K = int(0.7 * _C)

_HH_P = 56
_NH_P = _H // _HH_P
_HH = 28
_NH = _H // _HH


def _pool_body(x_ref, out_ref):
    out_ref[0, 0, 0, :] = jnp.sum(x_ref[0], axis=(0, 1))


def _mlp_mask_body(psums_ref, w1_ref, b1_ref, w2_ref, b2_ref,
                   soft_ref, mask_ref):
    pooled = jnp.sum(psums_ref[...], axis=(1, 2)) * (1.0 / _HW)
    h = lax.dot_general(pooled, w1_ref[...],
                        (((1,), (1,)), ((), ())),
                        preferred_element_type=jnp.float32)
    h = jnp.maximum(h + b1_ref[...][None, :], 0.0)
    z = lax.dot_general(h, w2_ref[...],
                        (((1,), (1,)), ((), ())),
                        preferred_element_type=jnp.float32)
    soft = jax.nn.sigmoid(z + b2_ref[...][None, :])
    soft_ref[...] = soft
    vi = soft[:, :, None]
    vj = soft[:, None, :]
    ii = lax.broadcasted_iota(jnp.int32, (_B, _C, _C), 1)
    jj = lax.broadcasted_iota(jnp.int32, (_B, _C, _C), 2)
    beats = (vj > vi) | ((vj == vi) & (jj < ii))
    rank = jnp.sum(beats.astype(jnp.int32), axis=2)
    mask_ref[...] = (rank < _K).astype(jnp.float32)


def _apply_body(mask_ref, x_ref, causal_ref, noncausal_ref):
    m = mask_ref[0, 0, :][None, None, :]
    x = x_ref[0]
    c = x * m
    causal_ref[0] = c
    noncausal_ref[0] = x - c


@jax.jit
def kernel(feat, w1, b1, w2, b2):
    x = jnp.transpose(feat, (0, 2, 3, 1))

    psums = pl.pallas_call(
        _pool_body,
        grid=(_B, _NH_P),
        in_specs=[
            pl.BlockSpec((1, _HH_P, _W, _C), lambda b, h: (b, h, 0, 0)),
        ],
        out_specs=pl.BlockSpec((1, 1, 1, _C), lambda b, h: (b, h, 0, 0)),
        out_shape=jax.ShapeDtypeStruct((_B, _NH_P, 1, _C), jnp.float32),
    )(x)

    soft_mask, mask = pl.pallas_call(
        _mlp_mask_body,
        out_shape=[
            jax.ShapeDtypeStruct((_B, _C), jnp.float32),
            jax.ShapeDtypeStruct((_B, _C), jnp.float32),
        ],
    )(psums, w1, b1, w2, b2)

    causal, noncausal = pl.pallas_call(
        _apply_body,
        grid=(_B, _NH),
        in_specs=[
            pl.BlockSpec((1, 1, _C), lambda b, h: (b, 0, 0)),
            pl.BlockSpec((1, _HH, _W, _C), lambda b, h: (b, h, 0, 0)),
        ],
        out_specs=[
            pl.BlockSpec((1, _HH, _W, _C), lambda b, h: (b, h, 0, 0)),
            pl.BlockSpec((1, _HH, _W, _C), lambda b, h: (b, h, 0, 0)),
        ],
        out_shape=[
            jax.ShapeDtypeStruct((_B, _H, _W, _C), jnp.float32),
            jax.ShapeDtypeStruct((_B, _H, _W, _C), jnp.float32),
        ],
    )(mask.reshape(_B, 1, _C), x)

    causal = jnp.transpose(causal, (0, 3, 1, 2))
    noncausal = jnp.transpose(noncausal, (0, 3, 1, 2))
    mask4 = mask.reshape(_B, _C, 1, 1)
    return (causal, noncausal, mask4, soft_mask)


# speedup vs baseline: 3.6222x; 1.0076x over previous
import jax
import jax.numpy as jnp
from jax import lax
from jax.experimental import pallas as pl

_B, _C, _H, _W = 4, 384, 224, 224
_HW = _H * _W
_K = int(0.7 * _C)

_HH_P = 56
_NH_P = _H // _HH_P
_HH = 28
_NH = _H // _HH


def _pool_body(x_ref, out_ref):
    out_ref[0, 0, 0, :] = jnp.sum(x_ref[0], axis=(0, 1))


def _apply_body(psums_ref, w1_ref, b1_ref, w2_ref, b2_ref, x_ref,
                causal_ref, noncausal_ref, soft_ref, mask_ref):
    b = pl.program_id(0)
    h = pl.program_id(1)

    @pl.when((b == 0) & (h == 0))
    def _compute_mask():
        pooled = jnp.sum(psums_ref[...], axis=(1, 2)) * (1.0 / _HW)
        hid = lax.dot_general(pooled, w1_ref[...],
                              (((1,), (1,)), ((), ())),
                              preferred_element_type=jnp.float32)
        hid = jnp.maximum(hid + b1_ref[...][None, :], 0.0)
        z = lax.dot_general(hid, w2_ref[...],
                            (((1,), (1,)), ((), ())),
                            preferred_element_type=jnp.float32)
        soft = jax.nn.sigmoid(z + b2_ref[...][None, :])
        soft_ref[...] = soft
        vi = soft[:, :, None]
        vj = soft[:, None, :]
        ii = lax.broadcasted_iota(jnp.int32, (_B, _C, _C), 1)
        jj = lax.broadcasted_iota(jnp.int32, (_B, _C, _C), 2)
        beats = (vj > vi) | ((vj == vi) & (jj < ii))
        rank = jnp.sum(beats.astype(jnp.int32), axis=2)
        mask_ref[...] = (rank < _K).astype(jnp.float32)

    m = mask_ref[b, :][None, None, :]
    x = x_ref[0]
    c = x * m
    causal_ref[0] = c
    noncausal_ref[0] = x - c


@jax.jit
def kernel(feat, w1, b1, w2, b2):
    x = jnp.transpose(feat, (0, 2, 3, 1))

    psums = pl.pallas_call(
        _pool_body,
        grid=(_B, _NH_P),
        in_specs=[
            pl.BlockSpec((1, _HH_P, _W, _C), lambda b, h: (b, h, 0, 0)),
        ],
        out_specs=pl.BlockSpec((1, 1, 1, _C), lambda b, h: (b, h, 0, 0)),
        out_shape=jax.ShapeDtypeStruct((_B, _NH_P, 1, _C), jnp.float32),
    )(x)

    causal, noncausal, soft_mask, mask = pl.pallas_call(
        _apply_body,
        grid=(_B, _NH),
        in_specs=[
            pl.BlockSpec((_B, _NH_P, 1, _C), lambda b, h: (0, 0, 0, 0)),
            pl.BlockSpec((384, _C), lambda b, h: (0, 0)),
            pl.BlockSpec((384,), lambda b, h: (0,)),
            pl.BlockSpec((_C, 384), lambda b, h: (0, 0)),
            pl.BlockSpec((_C,), lambda b, h: (0,)),
            pl.BlockSpec((1, _HH, _W, _C), lambda b, h: (b, h, 0, 0)),
        ],
        out_specs=[
            pl.BlockSpec((1, _HH, _W, _C), lambda b, h: (b, h, 0, 0)),
            pl.BlockSpec((1, _HH, _W, _C), lambda b, h: (b, h, 0, 0)),
            pl.BlockSpec((_B, _C), lambda b, h: (0, 0)),
            pl.BlockSpec((_B, _C), lambda b, h: (0, 0)),
        ],
        out_shape=[
            jax.ShapeDtypeStruct((_B, _H, _W, _C), jnp.float32),
            jax.ShapeDtypeStruct((_B, _H, _W, _C), jnp.float32),
            jax.ShapeDtypeStruct((_B, _C), jnp.float32),
            jax.ShapeDtypeStruct((_B, _C), jnp.float32),
        ],
    )(psums, w1, b1, w2, b2, x)

    causal = jnp.transpose(causal, (0, 3, 1, 2))
    noncausal = jnp.transpose(noncausal, (0, 3, 1, 2))
    mask4 = mask.reshape(_B, _C, 1, 1)
    return (causal, noncausal, mask4, soft_mask)


# speedup vs baseline: 3.6310x; 1.0024x over previous
import jax
import jax.numpy as jnp
from jax import lax
from jax.experimental import pallas as pl
from jax.experimental.pallas import tpu as pltpu

_B, _C, _H, _W = 4, 384, 224, 224
_HW = _H * _W
_K = int(0.7 * _C)

_HH = 28
_NH = _H // _HH


def _body(w1_ref, b1_ref, w2_ref, b2_ref, x_ref,
          causal_ref, noncausal_ref, soft_ref, mask_ref,
          acc_ref, msk_ref):
    b = pl.program_id(0)
    t = pl.program_id(1)

    @pl.when(t == 0)
    def _pool_init():
        acc_ref[...] = jnp.sum(x_ref[0], axis=(0, 1))[None, :]

    @pl.when((t > 0) & (t < _NH))
    def _pool_acc():
        acc_ref[...] += jnp.sum(x_ref[0], axis=(0, 1))[None, :]

    @pl.when(t == _NH)
    def _compute_mask():
        pooled = acc_ref[...] * (1.0 / _HW)
        hid = lax.dot_general(pooled, w1_ref[...],
                              (((1,), (1,)), ((), ())),
                              preferred_element_type=jnp.float32)
        hid = jnp.maximum(hid + b1_ref[...][None, :], 0.0)
        z = lax.dot_general(hid, w2_ref[...],
                            (((1,), (1,)), ((), ())),
                            preferred_element_type=jnp.float32)
        soft = jax.nn.sigmoid(z + b2_ref[...][None, :])
        soft_ref[b, :] = soft[0]
        vi = soft[0][:, None]
        vj = soft[0][None, :]
        ii = lax.broadcasted_iota(jnp.int32, (_C, _C), 0)
        jj = lax.broadcasted_iota(jnp.int32, (_C, _C), 1)
        beats = (vj > vi) | ((vj == vi) & (jj < ii))
        rank = jnp.sum(beats.astype(jnp.int32), axis=1)
        m = (rank < _K).astype(jnp.float32)
        msk_ref[...] = m[None, :]
        mask_ref[b, :] = m

    @pl.when(t >= _NH)
    def _apply():
        m = msk_ref[0][None, None, :]
        x = x_ref[0]
        c = x * m
        causal_ref[0] = c
        noncausal_ref[0] = x - c


@jax.jit
def kernel(feat, w1, b1, w2, b2):
    x = jnp.transpose(feat, (0, 2, 3, 1))

    def _xmap(b, t):
        return (b, jnp.where(t < _NH, t, t - _NH), 0, 0)

    def _omap(b, t):
        return (b, jnp.where(t < _NH, 0, t - _NH), 0, 0)

    causal, noncausal, soft_mask, mask = pl.pallas_call(
        _body,
        grid=(_B, 2 * _NH),
        in_specs=[
            pl.BlockSpec((384, _C), lambda b, t: (0, 0)),
            pl.BlockSpec((384,), lambda b, t: (0,)),
            pl.BlockSpec((_C, 384), lambda b, t: (0, 0)),
            pl.BlockSpec((_C,), lambda b, t: (0,)),
            pl.BlockSpec((1, _HH, _W, _C), _xmap),
        ],
        out_specs=[
            pl.BlockSpec((1, _HH, _W, _C), _omap),
            pl.BlockSpec((1, _HH, _W, _C), _omap),
            pl.BlockSpec((_B, _C), lambda b, t: (0, 0)),
            pl.BlockSpec((_B, _C), lambda b, t: (0, 0)),
        ],
        out_shape=[
            jax.ShapeDtypeStruct((_B, _H, _W, _C), jnp.float32),
            jax.ShapeDtypeStruct((_B, _H, _W, _C), jnp.float32),
            jax.ShapeDtypeStruct((_B, _C), jnp.float32),
            jax.ShapeDtypeStruct((_B, _C), jnp.float32),
        ],
        scratch_shapes=[
            pltpu.VMEM((1, _C), jnp.float32),
            pltpu.VMEM((1, _C), jnp.float32),
        ],
    )(w1, b1, w2, b2, x)

    causal = jnp.transpose(causal, (0, 3, 1, 2))
    noncausal = jnp.transpose(noncausal, (0, 3, 1, 2))
    mask4 = mask.reshape(_B, _C, 1, 1)
    return (causal, noncausal, mask4, soft_mask)
